# baseline (speedup 1.0000x reference)
import functools

import jax
import jax.numpy as jnp
from jax import lax
from jax.experimental import pallas as pl
from jax.experimental.pallas import tpu as pltpu, tpu_sc as plsc

N = 10000
E = 320000
D = 128
H = 256
G = 64
C = 10

NC = 2
NS = 16
NW = NC * NS
CHUNK = 128
NCHUNK = 80
EPW = NCHUNK * CHUNK
EPAD = NW * EPW
NBUF = 2
NSLOT = 4
NGRP = NCHUNK // NSLOT
NPAD = 10240
RPT = NPAD // NS
PSHIFT = 14
PMASK = (1 << PSHIFT) - 1


def _segsum_body(h_hbm, packed_hbm, zeros_hbm, out_hbm,
                 packed_v, srcb, dstb, rows_a, rows_b, shared, *sems):
    rows = (rows_a, rows_b)
    gsems = sems[:2]
    ssems = sems[2:]

    def unpack(j, q):
        for k in range(CHUNK // 16):
            v = packed_v[j, pl.ds(k * 16, 16)]
            srcb[q, pl.ds(k * 16, 16)] = lax.shift_right_logical(v, PSHIFT)
            dstb[q, pl.ds(k * 16, 16)] = lax.bitwise_and(v, PMASK)

    def gather(j_slot, b):
        return pltpu.async_copy(h_hbm.at[srcb.at[j_slot]], rows[b], gsems[b])

    def scatter(j_slot, b):
        return pltpu.async_copy(rows[b], shared.at[dstb.at[j_slot]],
                                ssems[b], add=True)

    c = lax.axis_index("c")
    s = lax.axis_index("s")
    w = c * NS + s
    pltpu.sync_copy(zeros_hbm, shared.at[pl.ds(s * RPT, RPT)])
    pltpu.sync_copy(packed_hbm.at[w], packed_v)
    for b in range(2):
        unpack(b, b)
        gather(b, b)
    plsc.subcore_barrier()

    def group(g, carry):
        for u in range(NSLOT):
            i = g * NSLOT + u
            b = u % 2
            ob = (u + 1) % 2
            q1 = (u + 1) % NSLOT
            qp = (u + 3) % NSLOT
            pltpu.make_async_copy(h_hbm.at[srcb.at[u]], rows[b],
                                  gsems[b]).wait()

            @pl.when(i >= 1)
            def _():
                pltpu.make_async_copy(rows[ob], shared.at[dstb.at[qp]],
                                      ssems[ob]).wait()

            scatter(u, b)

            @pl.when(i + 1 < NCHUNK)
            def _():
                unpack(i + 1, q1)
                gather(q1, ob)
        return carry

    lax.fori_loop(0, NGRP, group, 0)
    pltpu.make_async_copy(rows[(NCHUNK - 1) % 2],
                          shared.at[dstb.at[(NCHUNK - 1) % NSLOT]],
                          ssems[(NCHUNK - 1) % 2]).wait()
    plsc.subcore_barrier()
    pltpu.sync_copy(shared.at[pl.ds(s * RPT, RPT)],
                    out_hbm.at[c, pl.ds(s * RPT, RPT)])


def _segsum(h, packed, zeros):
    mesh = plsc.VectorSubcoreMesh(core_axis_name="c", subcore_axis_name="s")
    f = pl.kernel(
        _segsum_body,
        mesh=mesh,
        out_type=jax.ShapeDtypeStruct((NC, NPAD, D), jnp.float32),
        scratch_types=[
            pltpu.VMEM((NCHUNK, CHUNK), jnp.int32),
            pltpu.VMEM((NSLOT, CHUNK), jnp.int32),
            pltpu.VMEM((NSLOT, CHUNK), jnp.int32),
            pltpu.VMEM((CHUNK, D), jnp.float32),
            pltpu.VMEM((CHUNK, D), jnp.float32),
            pltpu.VMEM_SHARED((NPAD, D), jnp.float32),
        ] + [pltpu.SemaphoreType.DMA] * 4,
    )
    return f(h, packed, zeros)


BLK = 2000


def _mlp_compute(h_ref, a_ref, W1_ref, b1_ref, W2_ref, b2_ref, relu_out):
    z = h_ref[...] + a_ref[0] + a_ref[1]
    z = jnp.dot(z, W1_ref[...], preferred_element_type=jnp.float32,
                precision=lax.Precision.HIGHEST) + b1_ref[...]
    z = jnp.maximum(z, 0.0)
    z = jnp.dot(z, W2_ref[...], preferred_element_type=jnp.float32,
                precision=lax.Precision.HIGHEST) + b2_ref[...]
    if relu_out:
        z = jnp.maximum(z, 0.0)
    return z


def _mlp_body(h_ref, a_ref, W1_ref, b1_ref, W2_ref, b2_ref, o_ref):
    o_ref[...] = _mlp_compute(h_ref, a_ref, W1_ref, b1_ref, W2_ref, b2_ref,
                              relu_out=True)


def _mlp(h, agg, W1, b1, W2, b2):
    return pl.pallas_call(
        _mlp_body,
        grid=(N // BLK,),
        in_specs=[
            pl.BlockSpec((BLK, D), lambda i: (i, 0)),
            pl.BlockSpec((NC, BLK, D), lambda i: (0, i, 0)),
            pl.BlockSpec((D, H), lambda i: (0, 0)),
            pl.BlockSpec((1, H), lambda i: (0, 0)),
            pl.BlockSpec((H, D), lambda i: (0, 0)),
            pl.BlockSpec((1, D), lambda i: (0, 0)),
        ],
        out_specs=pl.BlockSpec((BLK, D), lambda i: (i, 0)),
        out_shape=jax.ShapeDtypeStruct((N, D), jnp.float32),
    )(h, agg, W1, b1, W2, b2)


def _mlp_pool_body(h_ref, a_ref, batch_ref, W1_ref, b1_ref, W2_ref, b2_ref,
                   cw_ref, cb_ref, o_ref, sums_ref, cnt_ref):
    i = pl.program_id(0)

    @pl.when(i == 0)
    def _():
        sums_ref[...] = jnp.zeros_like(sums_ref)
        cnt_ref[...] = jnp.zeros_like(cnt_ref)

    z = _mlp_compute(h_ref, a_ref, W1_ref, b1_ref, W2_ref, b2_ref,
                     relu_out=False)
    onehot = (batch_ref[...] ==
              lax.broadcasted_iota(jnp.int32, (BLK, G), 1)).astype(jnp.float32)
    dn = (((0,), (0,)), ((), ()))
    sums_ref[...] += lax.dot_general(onehot, z, dn,
                                     preferred_element_type=jnp.float32,
                                     precision=lax.Precision.HIGHEST)
    cnt_ref[...] += lax.dot_general(onehot, jnp.ones((BLK, D), jnp.float32),
                                    dn, preferred_element_type=jnp.float32,
                                    precision=lax.Precision.HIGHEST)

    @pl.when(i == pl.num_programs(0) - 1)
    def _():
        hg = sums_ref[...] / jnp.maximum(cnt_ref[...], 1.0)
        o_ref[...] = jnp.dot(hg, cw_ref[...],
                             preferred_element_type=jnp.float32,
                             precision=lax.Precision.HIGHEST) + cb_ref[...]


def _mlp_pool(h, agg, batch2, W1, b1, W2, b2, cls_W, cls_b2):
    return pl.pallas_call(
        _mlp_pool_body,
        grid=(N // BLK,),
        in_specs=[
            pl.BlockSpec((BLK, D), lambda i: (i, 0)),
            pl.BlockSpec((NC, BLK, D), lambda i: (0, i, 0)),
            pl.BlockSpec((BLK, 1), lambda i: (i, 0)),
            pl.BlockSpec((D, H), lambda i: (0, 0)),
            pl.BlockSpec((1, H), lambda i: (0, 0)),
            pl.BlockSpec((H, D), lambda i: (0, 0)),
            pl.BlockSpec((1, D), lambda i: (0, 0)),
            pl.BlockSpec((D, C), lambda i: (0, 0)),
            pl.BlockSpec((1, C), lambda i: (0, 0)),
        ],
        out_specs=pl.BlockSpec((G, C), lambda i: (0, 0)),
        out_shape=jax.ShapeDtypeStruct((G, C), jnp.float32),
        scratch_shapes=[
            pltpu.VMEM((G, D), jnp.float32),
            pltpu.VMEM((G, D), jnp.float32),
        ],
    )(h, agg, batch2, W1, b1, W2, b2, cls_W, cls_b2)


def kernel(x, edge_index, batch,
           l0_W1, l0_b1, l0_bn_g, l0_bn_b, l0_W2, l0_b2, l0_obn_g, l0_obn_b,
           l1_W1, l1_b1, l1_bn_g, l1_bn_b, l1_W2, l1_b2, l1_obn_g, l1_obn_b,
           l2_W1, l2_b1, l2_bn_g, l2_bn_b, l2_W2, l2_b2, l2_obn_g, l2_obn_b,
           cls_W, cls_b):
    bscale = 1.0 / jnp.sqrt(jnp.float32(1.0 + 1e-5))
    layers = [
        (l0_W1, l0_b1, l0_bn_g, l0_bn_b, l0_W2, l0_b2, l0_obn_g, l0_obn_b),
        (l1_W1, l1_b1, l1_bn_g, l1_bn_b, l1_W2, l1_b2, l1_obn_g, l1_obn_b),
        (l2_W1, l2_b1, l2_bn_g, l2_bn_b, l2_W2, l2_b2, l2_obn_g, l2_obn_b),
    ]
    folded = []
    for (W1, b1, bg, bb, W2, b2, og, ob) in layers:
        s1 = bscale * bg
        s2 = bscale * og
        folded.append((W1 * s1[None, :], (b1 * s1 + bb)[None, :],
                       W2 * s2[None, :], (b2 * s2 + ob)[None, :]))

    padw = EPW - E // NW
    pad_src = jnp.broadcast_to(jnp.arange(padw, dtype=jnp.int32), (NW, padw))
    pad_dst = jnp.broadcast_to(N + jnp.arange(padw, dtype=jnp.int32),
                               (NW, padw))
    srcp = jnp.concatenate([edge_index[0].reshape(NW, E // NW), pad_src], 1)
    dstp = jnp.concatenate([edge_index[1].reshape(NW, E // NW), pad_dst], 1)
    packed = ((srcp << PSHIFT) | dstp).reshape(NW, NCHUNK, CHUNK)
    zeros = jnp.zeros((RPT, D), jnp.float32)
    batch2 = batch.reshape(N, 1)

    h = x
    for l in range(2):
        W1f, b1f, W2f, b2f = folded[l]
        agg = _segsum(h, packed, zeros)
        h = _mlp(h, agg, W1f, b1f, W2f, b2f)
    W1f, b1f, W2f, b2f = folded[2]
    agg = _segsum(h, packed, zeros)
    return _mlp_pool(h, agg, batch2, W1f, b1f, W2f, b2f,
                     cls_W, cls_b.reshape(1, C))


# speedup vs baseline: 1.2479x; 1.2479x over previous
import functools

import jax
import jax.numpy as jnp
from jax import lax
from jax.experimental import pallas as pl
from jax.experimental.pallas import tpu as pltpu, tpu_sc as plsc

N = 10000
E = 320000
D = 128
H = 256
G = 64
C = 10

NC = 2
NS = 16
NW = NC * NS
CHUNK = 128
NCHUNK = 80
EPW = NCHUNK * CHUNK
EPAD = NW * EPW
NBUF = 2
NSLOT = 4
NGRP = NCHUNK // NSLOT
NPAD = 10240
RPT = NPAD // NS
PSHIFT = 14
PMASK = (1 << PSHIFT) - 1


def _segsum_body(h_hbm, packed_hbm, zeros_hbm, out_hbm,
                 packed_v, srcb, dstb, rows_a, rows_b, shared, *sems):
    rows = (rows_a, rows_b)
    gsems = sems

    def unpack(j, q):
        for k in range(CHUNK // 16):
            v = packed_v[j, pl.ds(k * 16, 16)]
            srcb[q, pl.ds(k * 16, 16)] = lax.shift_right_logical(v, PSHIFT)
            dstb[q, pl.ds(k * 16, 16)] = lax.bitwise_and(v, PMASK)

    def gather(j_slot, b):
        return pltpu.async_copy(h_hbm.at[srcb.at[j_slot]], rows[b], gsems[b])

    c = lax.axis_index("c")
    s = lax.axis_index("s")
    w = c * NS + s
    pltpu.sync_copy(zeros_hbm, shared.at[pl.ds(s * RPT, RPT)])
    pltpu.sync_copy(packed_hbm.at[w], packed_v)
    for b in range(2):
        unpack(b, b)
        gather(b, b)
    plsc.subcore_barrier()

    def group(g, carry):
        for u in range(NSLOT):
            i = g * NSLOT + u
            b = u % 2
            q2 = (u + 2) % NSLOT
            pltpu.make_async_copy(h_hbm.at[srcb.at[u]], rows[b],
                                  gsems[b]).wait()
            pltpu.sync_copy(rows[b], shared.at[dstb.at[u]], add=True)

            @pl.when(i + 2 < NCHUNK)
            def _():
                unpack(i + 2, q2)
                gather(q2, b)
        return carry

    lax.fori_loop(0, NGRP, group, 0)
    plsc.subcore_barrier()
    pltpu.sync_copy(shared.at[pl.ds(s * RPT, RPT)],
                    out_hbm.at[c, pl.ds(s * RPT, RPT)])


def _segsum(h, packed, zeros):
    mesh = plsc.VectorSubcoreMesh(core_axis_name="c", subcore_axis_name="s")
    f = pl.kernel(
        _segsum_body,
        mesh=mesh,
        out_type=jax.ShapeDtypeStruct((NC, NPAD, D), jnp.float32),
        scratch_types=[
            pltpu.VMEM((NCHUNK, CHUNK), jnp.int32),
            pltpu.VMEM((NSLOT, CHUNK), jnp.int32),
            pltpu.VMEM((NSLOT, CHUNK), jnp.int32),
            pltpu.VMEM((CHUNK, D), jnp.float32),
            pltpu.VMEM((CHUNK, D), jnp.float32),
            pltpu.VMEM_SHARED((NPAD, D), jnp.float32),
        ] + [pltpu.SemaphoreType.DMA] * 2,
    )
    return f(h, packed, zeros)


BLK = 2000


def _mlp_compute(h_ref, a_ref, W1_ref, b1_ref, W2_ref, b2_ref, relu_out):
    z = h_ref[...] + a_ref[0] + a_ref[1]
    z = jnp.dot(z, W1_ref[...], preferred_element_type=jnp.float32) + b1_ref[...]
    z = jnp.maximum(z, 0.0)
    z = jnp.dot(z, W2_ref[...], preferred_element_type=jnp.float32) + b2_ref[...]
    if relu_out:
        z = jnp.maximum(z, 0.0)
    return z


def _mlp_body(h_ref, a_ref, W1_ref, b1_ref, W2_ref, b2_ref, o_ref):
    o_ref[...] = _mlp_compute(h_ref, a_ref, W1_ref, b1_ref, W2_ref, b2_ref,
                              relu_out=True)


def _mlp(h, agg, W1, b1, W2, b2):
    return pl.pallas_call(
        _mlp_body,
        grid=(N // BLK,),
        in_specs=[
            pl.BlockSpec((BLK, D), lambda i: (i, 0)),
            pl.BlockSpec((NC, BLK, D), lambda i: (0, i, 0)),
            pl.BlockSpec((D, H), lambda i: (0, 0)),
            pl.BlockSpec((1, H), lambda i: (0, 0)),
            pl.BlockSpec((H, D), lambda i: (0, 0)),
            pl.BlockSpec((1, D), lambda i: (0, 0)),
        ],
        out_specs=pl.BlockSpec((BLK, D), lambda i: (i, 0)),
        out_shape=jax.ShapeDtypeStruct((N, D), jnp.float32),
    )(h, agg, W1, b1, W2, b2)


def _mlp_pool_body(h_ref, a_ref, batch_ref, W1_ref, b1_ref, W2_ref, b2_ref,
                   cw_ref, cb_ref, o_ref, sums_ref, cnt_ref):
    i = pl.program_id(0)

    @pl.when(i == 0)
    def _():
        sums_ref[...] = jnp.zeros_like(sums_ref)
        cnt_ref[...] = jnp.zeros_like(cnt_ref)

    z = _mlp_compute(h_ref, a_ref, W1_ref, b1_ref, W2_ref, b2_ref,
                     relu_out=False)
    onehot = (batch_ref[...] ==
              lax.broadcasted_iota(jnp.int32, (BLK, G), 1)).astype(jnp.float32)
    dn = (((0,), (0,)), ((), ()))
    sums_ref[...] += lax.dot_general(onehot, z, dn,
                                     preferred_element_type=jnp.float32)
    cnt_ref[...] += lax.dot_general(onehot, jnp.ones((BLK, D), jnp.float32),
                                    dn, preferred_element_type=jnp.float32)

    @pl.when(i == pl.num_programs(0) - 1)
    def _():
        hg = sums_ref[...] / jnp.maximum(cnt_ref[...], 1.0)
        o_ref[...] = jnp.dot(hg, cw_ref[...],
                             preferred_element_type=jnp.float32) + cb_ref[...]


def _mlp_pool(h, agg, batch2, W1, b1, W2, b2, cls_W, cls_b2):
    return pl.pallas_call(
        _mlp_pool_body,
        grid=(N // BLK,),
        in_specs=[
            pl.BlockSpec((BLK, D), lambda i: (i, 0)),
            pl.BlockSpec((NC, BLK, D), lambda i: (0, i, 0)),
            pl.BlockSpec((BLK, 1), lambda i: (i, 0)),
            pl.BlockSpec((D, H), lambda i: (0, 0)),
            pl.BlockSpec((1, H), lambda i: (0, 0)),
            pl.BlockSpec((H, D), lambda i: (0, 0)),
            pl.BlockSpec((1, D), lambda i: (0, 0)),
            pl.BlockSpec((D, C), lambda i: (0, 0)),
            pl.BlockSpec((1, C), lambda i: (0, 0)),
        ],
        out_specs=pl.BlockSpec((G, C), lambda i: (0, 0)),
        out_shape=jax.ShapeDtypeStruct((G, C), jnp.float32),
        scratch_shapes=[
            pltpu.VMEM((G, D), jnp.float32),
            pltpu.VMEM((G, D), jnp.float32),
        ],
    )(h, agg, batch2, W1, b1, W2, b2, cls_W, cls_b2)


def kernel(x, edge_index, batch,
           l0_W1, l0_b1, l0_bn_g, l0_bn_b, l0_W2, l0_b2, l0_obn_g, l0_obn_b,
           l1_W1, l1_b1, l1_bn_g, l1_bn_b, l1_W2, l1_b2, l1_obn_g, l1_obn_b,
           l2_W1, l2_b1, l2_bn_g, l2_bn_b, l2_W2, l2_b2, l2_obn_g, l2_obn_b,
           cls_W, cls_b):
    bscale = 1.0 / jnp.sqrt(jnp.float32(1.0 + 1e-5))
    layers = [
        (l0_W1, l0_b1, l0_bn_g, l0_bn_b, l0_W2, l0_b2, l0_obn_g, l0_obn_b),
        (l1_W1, l1_b1, l1_bn_g, l1_bn_b, l1_W2, l1_b2, l1_obn_g, l1_obn_b),
        (l2_W1, l2_b1, l2_bn_g, l2_bn_b, l2_W2, l2_b2, l2_obn_g, l2_obn_b),
    ]
    folded = []
    for (W1, b1, bg, bb, W2, b2, og, ob) in layers:
        s1 = bscale * bg
        s2 = bscale * og
        folded.append((W1 * s1[None, :], (b1 * s1 + bb)[None, :],
                       W2 * s2[None, :], (b2 * s2 + ob)[None, :]))

    padw = EPW - E // NW
    pad_src = jnp.broadcast_to(jnp.arange(padw, dtype=jnp.int32), (NW, padw))
    pad_dst = jnp.broadcast_to(N + jnp.arange(padw, dtype=jnp.int32),
                               (NW, padw))
    srcp = jnp.concatenate([edge_index[0].reshape(NW, E // NW), pad_src], 1)
    dstp = jnp.concatenate([edge_index[1].reshape(NW, E // NW), pad_dst], 1)
    packed = ((srcp << PSHIFT) | dstp).reshape(NW, NCHUNK, CHUNK)
    zeros = jnp.zeros((RPT, D), jnp.float32)
    batch2 = batch.reshape(N, 1)

    h = x
    for l in range(2):
        W1f, b1f, W2f, b2f = folded[l]
        agg = _segsum(h, packed, zeros)
        h = _mlp(h, agg, W1f, b1f, W2f, b2f)
    W1f, b1f, W2f, b2f = folded[2]
    agg = _segsum(h, packed, zeros)
    return _mlp_pool(h, agg, batch2, W1f, b1f, W2f, b2f,
                     cls_W, cls_b.reshape(1, C))


# speedup vs baseline: 1.2588x; 1.0088x over previous
import functools

import jax
import jax.numpy as jnp
from jax import lax
from jax.experimental import pallas as pl
from jax.experimental.pallas import tpu as pltpu, tpu_sc as plsc

N = 10000
E = 320000
D = 128
H = 256
G = 64
C = 10

NC = 2
NS = 16
NW = NC * NS
CHUNK = 128
NCHUNK = 80
EPW = NCHUNK * CHUNK
EPAD = NW * EPW
NBUF = 2
NSLOT = 4
NGRP = NCHUNK // NSLOT
NPAD = 10240
RPT = NPAD // NS
PSHIFT = 14
PMASK = (1 << PSHIFT) - 1


def _segsum_body(h_hbm, packed_hbm, zeros_hbm, out_hbm,
                 packed_v, srcb, dstb, rows_a, rows_b, shared, *sems):
    rows = (rows_a, rows_b)
    gsems = sems

    def unpack(j, q):
        for k in range(CHUNK // 16):
            v = packed_v[j, pl.ds(k * 16, 16)]
            srcb[q, pl.ds(k * 16, 16)] = lax.shift_right_logical(v, PSHIFT)
            dstb[q, pl.ds(k * 16, 16)] = lax.bitwise_and(v, PMASK)

    def gather(j_slot, b):
        return pltpu.async_copy(h_hbm.at[srcb.at[j_slot]], rows[b], gsems[b])

    c = lax.axis_index("c")
    s = lax.axis_index("s")
    w = c * NS + s
    pltpu.sync_copy(zeros_hbm, shared.at[pl.ds(s * RPT, RPT)])
    pltpu.sync_copy(packed_hbm.at[w], packed_v)
    for b in range(2):
        unpack(b, b)
        gather(b, b)
    plsc.subcore_barrier()

    def group(g, carry):
        for u in range(NSLOT):
            i = g * NSLOT + u
            b = u % 2
            q2 = (u + 2) % NSLOT
            pltpu.make_async_copy(h_hbm.at[srcb.at[u]], rows[b],
                                  gsems[b]).wait()
            pltpu.sync_copy(rows[b], shared.at[dstb.at[u]], add=True)

            @pl.when(i + 2 < NCHUNK)
            def _():
                unpack(i + 2, q2)
                gather(q2, b)
        return carry

    lax.fori_loop(0, NGRP, group, 0)
    plsc.subcore_barrier()
    pltpu.sync_copy(shared.at[pl.ds(s * RPT, RPT)],
                    out_hbm.at[c, pl.ds(s * RPT, RPT)])


def _segsum(h, packed, zeros):
    mesh = plsc.VectorSubcoreMesh(core_axis_name="c", subcore_axis_name="s")
    f = pl.kernel(
        _segsum_body,
        mesh=mesh,
        out_type=jax.ShapeDtypeStruct((NC, NPAD, D), jnp.float32),
        scratch_types=[
            pltpu.VMEM((NCHUNK, CHUNK), jnp.int32),
            pltpu.VMEM((NSLOT, CHUNK), jnp.int32),
            pltpu.VMEM((NSLOT, CHUNK), jnp.int32),
            pltpu.VMEM((CHUNK, D), jnp.float32),
            pltpu.VMEM((CHUNK, D), jnp.float32),
            pltpu.VMEM_SHARED((NPAD, D), jnp.float32),
        ] + [pltpu.SemaphoreType.DMA] * 2,
    )
    return f(h, packed, zeros)


BLK = 5000


def _mlp_compute(h_ref, a_ref, W1_ref, b1_ref, W2_ref, b2_ref, relu_out):
    z = h_ref[...] + a_ref[0] + a_ref[1]
    z = jnp.dot(z, W1_ref[...], preferred_element_type=jnp.float32) + b1_ref[...]
    z = jnp.maximum(z, 0.0)
    z = jnp.dot(z, W2_ref[...], preferred_element_type=jnp.float32) + b2_ref[...]
    if relu_out:
        z = jnp.maximum(z, 0.0)
    return z


def _mlp_body(h_ref, a_ref, W1_ref, b1_ref, W2_ref, b2_ref, o_ref):
    o_ref[...] = _mlp_compute(h_ref, a_ref, W1_ref, b1_ref, W2_ref, b2_ref,
                              relu_out=True)


def _mlp(h, agg, W1, b1, W2, b2):
    return pl.pallas_call(
        _mlp_body,
        grid=(N // BLK,),
        in_specs=[
            pl.BlockSpec((BLK, D), lambda i: (i, 0)),
            pl.BlockSpec((NC, BLK, D), lambda i: (0, i, 0)),
            pl.BlockSpec((D, H), lambda i: (0, 0)),
            pl.BlockSpec((1, H), lambda i: (0, 0)),
            pl.BlockSpec((H, D), lambda i: (0, 0)),
            pl.BlockSpec((1, D), lambda i: (0, 0)),
        ],
        out_specs=pl.BlockSpec((BLK, D), lambda i: (i, 0)),
        out_shape=jax.ShapeDtypeStruct((N, D), jnp.float32),
    )(h, agg, W1, b1, W2, b2)


def _mlp_pool_body(h_ref, a_ref, batch_ref, W1_ref, b1_ref, W2_ref, b2_ref,
                   cw_ref, cb_ref, o_ref, sums_ref, cnt_ref):
    i = pl.program_id(0)

    @pl.when(i == 0)
    def _():
        sums_ref[...] = jnp.zeros_like(sums_ref)
        cnt_ref[...] = jnp.zeros_like(cnt_ref)

    z = _mlp_compute(h_ref, a_ref, W1_ref, b1_ref, W2_ref, b2_ref,
                     relu_out=False)
    onehot = (batch_ref[...] ==
              lax.broadcasted_iota(jnp.int32, (BLK, G), 1)).astype(jnp.float32)
    dn = (((0,), (0,)), ((), ()))
    sums_ref[...] += lax.dot_general(onehot, z, dn,
                                     preferred_element_type=jnp.float32)
    cnt_ref[...] += lax.dot_general(onehot, jnp.ones((BLK, D), jnp.float32),
                                    dn, preferred_element_type=jnp.float32)

    @pl.when(i == pl.num_programs(0) - 1)
    def _():
        hg = sums_ref[...] / jnp.maximum(cnt_ref[...], 1.0)
        o_ref[...] = jnp.dot(hg, cw_ref[...],
                             preferred_element_type=jnp.float32) + cb_ref[...]


def _mlp_pool(h, agg, batch2, W1, b1, W2, b2, cls_W, cls_b2):
    return pl.pallas_call(
        _mlp_pool_body,
        grid=(N // BLK,),
        in_specs=[
            pl.BlockSpec((BLK, D), lambda i: (i, 0)),
            pl.BlockSpec((NC, BLK, D), lambda i: (0, i, 0)),
            pl.BlockSpec((BLK, 1), lambda i: (i, 0)),
            pl.BlockSpec((D, H), lambda i: (0, 0)),
            pl.BlockSpec((1, H), lambda i: (0, 0)),
            pl.BlockSpec((H, D), lambda i: (0, 0)),
            pl.BlockSpec((1, D), lambda i: (0, 0)),
            pl.BlockSpec((D, C), lambda i: (0, 0)),
            pl.BlockSpec((1, C), lambda i: (0, 0)),
        ],
        out_specs=pl.BlockSpec((G, C), lambda i: (0, 0)),
        out_shape=jax.ShapeDtypeStruct((G, C), jnp.float32),
        scratch_shapes=[
            pltpu.VMEM((G, D), jnp.float32),
            pltpu.VMEM((G, D), jnp.float32),
        ],
    )(h, agg, batch2, W1, b1, W2, b2, cls_W, cls_b2)


def kernel(x, edge_index, batch,
           l0_W1, l0_b1, l0_bn_g, l0_bn_b, l0_W2, l0_b2, l0_obn_g, l0_obn_b,
           l1_W1, l1_b1, l1_bn_g, l1_bn_b, l1_W2, l1_b2, l1_obn_g, l1_obn_b,
           l2_W1, l2_b1, l2_bn_g, l2_bn_b, l2_W2, l2_b2, l2_obn_g, l2_obn_b,
           cls_W, cls_b):
    bscale = 1.0 / jnp.sqrt(jnp.float32(1.0 + 1e-5))
    layers = [
        (l0_W1, l0_b1, l0_bn_g, l0_bn_b, l0_W2, l0_b2, l0_obn_g, l0_obn_b),
        (l1_W1, l1_b1, l1_bn_g, l1_bn_b, l1_W2, l1_b2, l1_obn_g, l1_obn_b),
        (l2_W1, l2_b1, l2_bn_g, l2_bn_b, l2_W2, l2_b2, l2_obn_g, l2_obn_b),
    ]
    folded = []
    for (W1, b1, bg, bb, W2, b2, og, ob) in layers:
        s1 = bscale * bg
        s2 = bscale * og
        folded.append((W1 * s1[None, :], (b1 * s1 + bb)[None, :],
                       W2 * s2[None, :], (b2 * s2 + ob)[None, :]))

    padw = EPW - E // NW
    pad_src = jnp.broadcast_to(jnp.arange(padw, dtype=jnp.int32), (NW, padw))
    pad_dst = jnp.broadcast_to(N + jnp.arange(padw, dtype=jnp.int32),
                               (NW, padw))
    srcp = jnp.concatenate([edge_index[0].reshape(NW, E // NW), pad_src], 1)
    dstp = jnp.concatenate([edge_index[1].reshape(NW, E // NW), pad_dst], 1)
    packed = ((srcp << PSHIFT) | dstp).reshape(NW, NCHUNK, CHUNK)
    zeros = jnp.zeros((RPT, D), jnp.float32)
    batch2 = batch.reshape(N, 1)

    h = x
    for l in range(2):
        W1f, b1f, W2f, b2f = folded[l]
        agg = _segsum(h, packed, zeros)
        h = _mlp(h, agg, W1f, b1f, W2f, b2f)
    W1f, b1f, W2f, b2f = folded[2]
    agg = _segsum(h, packed, zeros)
    return _mlp_pool(h, agg, batch2, W1f, b1f, W2f, b2f,
                     cls_W, cls_b.reshape(1, C))


# speedup vs baseline: 1.2855x; 1.0212x over previous
import functools

import jax
import jax.numpy as jnp
from jax import lax
from jax.experimental import pallas as pl
from jax.experimental.pallas import tpu as pltpu, tpu_sc as plsc

N = 10000
E = 320000
D = 128
H = 256
G = 64
C = 10

NC = 2
NS = 16
NW = NC * NS
CHUNK = 128
NCHUNK = 80
EPW = NCHUNK * CHUNK
EPAD = NW * EPW
NBUF = 2
NSLOT = 4
NGRP = NCHUNK // NSLOT
NPAD = 10240
RPT = NPAD // NS
PSHIFT = 14
PMASK = (1 << PSHIFT) - 1


def _segsum_body(h_hbm, packed_hbm, zeros_hbm, out_hbm,
                 packed_v, srcb, dstb, rows_a, rows_b, shared, *sems):
    rows = (rows_a, rows_b)
    gsems = sems

    def unpack(j, q):
        for k in range(CHUNK // 16):
            v = packed_v[j, pl.ds(k * 16, 16)]
            srcb[q, pl.ds(k * 16, 16)] = lax.shift_right_logical(v, PSHIFT)
            dstb[q, pl.ds(k * 16, 16)] = lax.bitwise_and(v, PMASK)

    def gather(j_slot, b):
        return pltpu.async_copy(h_hbm.at[srcb.at[j_slot]], rows[b], gsems[b])

    c = lax.axis_index("c")
    s = lax.axis_index("s")
    w = c * NS + s
    pltpu.sync_copy(packed_hbm.at[w], packed_v)
    for b in range(2):
        unpack(b, b)
        gather(b, b)
    pltpu.sync_copy(zeros_hbm.at[s], shared.at[pl.ds(s * RPT, RPT)])
    plsc.subcore_barrier()

    def group(g, carry):
        for u in range(NSLOT):
            i = g * NSLOT + u
            b = u % 2
            q2 = (u + 2) % NSLOT
            pltpu.make_async_copy(h_hbm.at[srcb.at[u]], rows[b],
                                  gsems[b]).wait()
            pltpu.sync_copy(rows[b], shared.at[dstb.at[u]], add=True)

            @pl.when(i + 2 < NCHUNK)
            def _():
                unpack(i + 2, q2)
                gather(q2, b)
        return carry

    lax.fori_loop(0, NGRP, group, 0)
    plsc.subcore_barrier()
    pltpu.sync_copy(shared.at[pl.ds(s * RPT, RPT)],
                    out_hbm.at[c, pl.ds(s * RPT, RPT)])


def _segsum(h, packed, zeros):
    mesh = plsc.VectorSubcoreMesh(core_axis_name="c", subcore_axis_name="s")
    f = pl.kernel(
        _segsum_body,
        mesh=mesh,
        out_type=jax.ShapeDtypeStruct((NC, NPAD, D), jnp.float32),
        scratch_types=[
            pltpu.VMEM((NCHUNK, CHUNK), jnp.int32),
            pltpu.VMEM((NSLOT, CHUNK), jnp.int32),
            pltpu.VMEM((NSLOT, CHUNK), jnp.int32),
            pltpu.VMEM((CHUNK, D), jnp.float32),
            pltpu.VMEM((CHUNK, D), jnp.float32),
            pltpu.VMEM_SHARED((NPAD, D), jnp.float32),
        ] + [pltpu.SemaphoreType.DMA] * 2,
    )
    return f(h, packed, zeros)


BLK = 5000


def _mlp_compute(h_ref, a_ref, W1_ref, b1_ref, W2_ref, b2_ref, relu_out):
    z = h_ref[...] + a_ref[0] + a_ref[1]
    z = jnp.dot(z, W1_ref[...], preferred_element_type=jnp.float32) + b1_ref[...]
    z = jnp.maximum(z, 0.0)
    z = jnp.dot(z, W2_ref[...], preferred_element_type=jnp.float32) + b2_ref[...]
    if relu_out:
        z = jnp.maximum(z, 0.0)
    return z


def _mlp_body(h_ref, a_ref, W1_ref, b1_ref, W2_ref, b2_ref, o_ref):
    o_ref[...] = _mlp_compute(h_ref, a_ref, W1_ref, b1_ref, W2_ref, b2_ref,
                              relu_out=True)


def _mlp(h, agg, W1, b1, W2, b2):
    return pl.pallas_call(
        _mlp_body,
        grid=(N // BLK,),
        in_specs=[
            pl.BlockSpec((BLK, D), lambda i: (i, 0)),
            pl.BlockSpec((NC, BLK, D), lambda i: (0, i, 0)),
            pl.BlockSpec((D, H), lambda i: (0, 0)),
            pl.BlockSpec((1, H), lambda i: (0, 0)),
            pl.BlockSpec((H, D), lambda i: (0, 0)),
            pl.BlockSpec((1, D), lambda i: (0, 0)),
        ],
        out_specs=pl.BlockSpec((BLK, D), lambda i: (i, 0)),
        out_shape=jax.ShapeDtypeStruct((N, D), jnp.float32),
    )(h, agg, W1, b1, W2, b2)


def _mlp_pool_body(h_ref, a_ref, batch_ref, W1_ref, b1_ref, W2_ref, b2_ref,
                   cw_ref, cb_ref, o_ref, sums_ref, cnt_ref):
    i = pl.program_id(0)

    @pl.when(i == 0)
    def _():
        sums_ref[...] = jnp.zeros_like(sums_ref)
        cnt_ref[...] = jnp.zeros_like(cnt_ref)

    z = _mlp_compute(h_ref, a_ref, W1_ref, b1_ref, W2_ref, b2_ref,
                     relu_out=False)
    onehot = (batch_ref[...] ==
              lax.broadcasted_iota(jnp.int32, (BLK, G), 1)).astype(jnp.float32)
    dn = (((0,), (0,)), ((), ()))
    sums_ref[...] += lax.dot_general(onehot, z, dn,
                                     preferred_element_type=jnp.float32)
    cnt_ref[...] += lax.dot_general(onehot, jnp.ones((BLK, D), jnp.float32),
                                    dn, preferred_element_type=jnp.float32)

    @pl.when(i == pl.num_programs(0) - 1)
    def _():
        hg = sums_ref[...] / jnp.maximum(cnt_ref[...], 1.0)
        o_ref[...] = jnp.dot(hg, cw_ref[...],
                             preferred_element_type=jnp.float32) + cb_ref[...]


def _mlp_pool(h, agg, batch2, W1, b1, W2, b2, cls_W, cls_b2):
    return pl.pallas_call(
        _mlp_pool_body,
        grid=(N // BLK,),
        in_specs=[
            pl.BlockSpec((BLK, D), lambda i: (i, 0)),
            pl.BlockSpec((NC, BLK, D), lambda i: (0, i, 0)),
            pl.BlockSpec((BLK, 1), lambda i: (i, 0)),
            pl.BlockSpec((D, H), lambda i: (0, 0)),
            pl.BlockSpec((1, H), lambda i: (0, 0)),
            pl.BlockSpec((H, D), lambda i: (0, 0)),
            pl.BlockSpec((1, D), lambda i: (0, 0)),
            pl.BlockSpec((D, C), lambda i: (0, 0)),
            pl.BlockSpec((1, C), lambda i: (0, 0)),
        ],
        out_specs=pl.BlockSpec((G, C), lambda i: (0, 0)),
        out_shape=jax.ShapeDtypeStruct((G, C), jnp.float32),
        scratch_shapes=[
            pltpu.VMEM((G, D), jnp.float32),
            pltpu.VMEM((G, D), jnp.float32),
        ],
    )(h, agg, batch2, W1, b1, W2, b2, cls_W, cls_b2)


def kernel(x, edge_index, batch,
           l0_W1, l0_b1, l0_bn_g, l0_bn_b, l0_W2, l0_b2, l0_obn_g, l0_obn_b,
           l1_W1, l1_b1, l1_bn_g, l1_bn_b, l1_W2, l1_b2, l1_obn_g, l1_obn_b,
           l2_W1, l2_b1, l2_bn_g, l2_bn_b, l2_W2, l2_b2, l2_obn_g, l2_obn_b,
           cls_W, cls_b):
    bscale = 1.0 / jnp.sqrt(jnp.float32(1.0 + 1e-5))
    layers = [
        (l0_W1, l0_b1, l0_bn_g, l0_bn_b, l0_W2, l0_b2, l0_obn_g, l0_obn_b),
        (l1_W1, l1_b1, l1_bn_g, l1_bn_b, l1_W2, l1_b2, l1_obn_g, l1_obn_b),
        (l2_W1, l2_b1, l2_bn_g, l2_bn_b, l2_W2, l2_b2, l2_obn_g, l2_obn_b),
    ]
    folded = []
    for (W1, b1, bg, bb, W2, b2, og, ob) in layers:
        s1 = bscale * bg
        s2 = bscale * og
        folded.append((W1 * s1[None, :], (b1 * s1 + bb)[None, :],
                       W2 * s2[None, :], (b2 * s2 + ob)[None, :]))

    padw = EPW - E // NW
    pad_src = jnp.broadcast_to(jnp.arange(padw, dtype=jnp.int32), (NW, padw))
    pad_dst = jnp.broadcast_to(N + jnp.arange(padw, dtype=jnp.int32),
                               (NW, padw))
    srcp = jnp.concatenate([edge_index[0].reshape(NW, E // NW), pad_src], 1)
    dstp = jnp.concatenate([edge_index[1].reshape(NW, E // NW), pad_dst], 1)
    packed = ((srcp << PSHIFT) | dstp).reshape(NW, NCHUNK, CHUNK)
    zeros = jnp.zeros((NS, RPT, D), jnp.float32)
    batch2 = batch.reshape(N, 1)

    h = x
    for l in range(2):
        W1f, b1f, W2f, b2f = folded[l]
        agg = _segsum(h, packed, zeros)
        h = _mlp(h, agg, W1f, b1f, W2f, b2f)
    W1f, b1f, W2f, b2f = folded[2]
    agg = _segsum(h, packed, zeros)
    return _mlp_pool(h, agg, batch2, W1f, b1f, W2f, b2f,
                     cls_W, cls_b.reshape(1, C))


# speedup vs baseline: 1.2894x; 1.0031x over previous
import jax
import jax.numpy as jnp
from jax import lax
from jax.experimental import pallas as pl
from jax.experimental.pallas import tpu as pltpu, tpu_sc as plsc

N = 10000
E = 320000
D = 128
H = 256
G = 64
C = 10

NC = 2
NS = 16
NW = NC * NS
CHUNK = 128
NCHUNK = 80
EPW = NCHUNK * CHUNK
EPAD = NW * EPW
NSLOT = 4
NGRP = NCHUNK // NSLOT
NPAD = 10240
RPT = NPAD // NS
PSHIFT = 14
PMASK = (1 << PSHIFT) - 1


def _segsum_body(h_hbm, packed_hbm, zeros_hbm, out_hbm,
                 packed_v, srcb, dstb, rows_a, rows_b, shared, *sems):
    rows = (rows_a, rows_b)
    gsems = sems

    def unpack(j, q):
        for k in range(CHUNK // 16):
            v = packed_v[j, pl.ds(k * 16, 16)]
            srcb[q, pl.ds(k * 16, 16)] = lax.shift_right_logical(v, PSHIFT)
            dstb[q, pl.ds(k * 16, 16)] = lax.bitwise_and(v, PMASK)

    def gather(j_slot, b):
        return pltpu.async_copy(h_hbm.at[srcb.at[j_slot]], rows[b], gsems[b])

    c = lax.axis_index("c")
    s = lax.axis_index("s")
    w = c * NS + s
    pltpu.sync_copy(packed_hbm.at[w], packed_v)
    for b in range(2):
        unpack(b, b)
        gather(b, b)
    pltpu.sync_copy(zeros_hbm.at[s], shared.at[pl.ds(s * RPT, RPT)])
    plsc.subcore_barrier()

    def group(g, carry):
        for u in range(NSLOT):
            i = g * NSLOT + u
            b = u % 2
            q2 = (u + 2) % NSLOT
            pltpu.make_async_copy(h_hbm.at[srcb.at[u]], rows[b],
                                  gsems[b]).wait()
            pltpu.sync_copy(rows[b], shared.at[dstb.at[u]], add=True)

            @pl.when(i + 2 < NCHUNK)
            def _():
                unpack(i + 2, q2)
                gather(q2, b)
        return carry

    lax.fori_loop(0, NGRP, group, 0)
    plsc.subcore_barrier()
    pltpu.sync_copy(shared.at[pl.ds(s * RPT, RPT)],
                    out_hbm.at[c, pl.ds(s * RPT, RPT)])


def _segsum(h, packed, zeros):
    mesh = plsc.VectorSubcoreMesh(core_axis_name="c", subcore_axis_name="s")
    f = pl.kernel(
        _segsum_body,
        mesh=mesh,
        out_type=jax.ShapeDtypeStruct((NC, NPAD, D), jnp.float32),
        scratch_types=[
            pltpu.VMEM((NCHUNK, CHUNK), jnp.int32),
            pltpu.VMEM((NSLOT, CHUNK), jnp.int32),
            pltpu.VMEM((NSLOT, CHUNK), jnp.int32),
            pltpu.VMEM((CHUNK, D), jnp.float32),
            pltpu.VMEM((CHUNK, D), jnp.float32),
            pltpu.VMEM_SHARED((NPAD, D), jnp.float32),
        ] + [pltpu.SemaphoreType.DMA] * 2,
    )
    return f(h, packed, zeros)


BLK = 5000


def _mlp_compute(h_ref, a_ref, W1_ref, b1_ref, W2_ref, b2_ref, relu_out):
    z = h_ref[...] + a_ref[0] + a_ref[1]
    z = jnp.dot(z, W1_ref[...], preferred_element_type=jnp.float32) + b1_ref[...]
    z = jnp.maximum(z, 0.0)
    z = jnp.dot(z, W2_ref[...], preferred_element_type=jnp.float32) + b2_ref[...]
    if relu_out:
        z = jnp.maximum(z, 0.0)
    return z


def _mlp_body(h_ref, a_ref, W1_ref, b1_ref, W2_ref, b2_ref, o_ref):
    o_ref[...] = _mlp_compute(h_ref, a_ref, W1_ref, b1_ref, W2_ref, b2_ref,
                              relu_out=True)


def _mlp(h, agg, W1, b1, W2, b2):
    return pl.pallas_call(
        _mlp_body,
        grid=(N // BLK,),
        in_specs=[
            pl.BlockSpec((BLK, D), lambda i: (i, 0)),
            pl.BlockSpec((NC, BLK, D), lambda i: (0, i, 0)),
            pl.BlockSpec((D, H), lambda i: (0, 0)),
            pl.BlockSpec((1, H), lambda i: (0, 0)),
            pl.BlockSpec((H, D), lambda i: (0, 0)),
            pl.BlockSpec((1, D), lambda i: (0, 0)),
        ],
        out_specs=pl.BlockSpec((BLK, D), lambda i: (i, 0)),
        out_shape=jax.ShapeDtypeStruct((N, D), jnp.float32),
    )(h, agg, W1, b1, W2, b2)


def _mlp_pool_body(h_ref, a_ref, batch_ref, W1_ref, b1_ref, W2_ref, b2_ref,
                   cw_ref, cb_ref, o_ref, sums_ref, cnt_ref):
    i = pl.program_id(0)

    @pl.when(i == 0)
    def _():
        sums_ref[...] = jnp.zeros_like(sums_ref)
        cnt_ref[...] = jnp.zeros_like(cnt_ref)

    z = _mlp_compute(h_ref, a_ref, W1_ref, b1_ref, W2_ref, b2_ref,
                     relu_out=False)
    onehot = (batch_ref[...] ==
              lax.broadcasted_iota(jnp.int32, (BLK, G), 1)).astype(jnp.float32)
    dn = (((0,), (0,)), ((), ()))
    sums_ref[...] += lax.dot_general(onehot, z, dn,
                                     preferred_element_type=jnp.float32)
    cnt_ref[...] += lax.dot_general(onehot, jnp.ones((BLK, D), jnp.float32),
                                    dn, preferred_element_type=jnp.float32)

    @pl.when(i == pl.num_programs(0) - 1)
    def _():
        hg = sums_ref[...] / jnp.maximum(cnt_ref[...], 1.0)
        o_ref[...] = jnp.dot(hg, cw_ref[...],
                             preferred_element_type=jnp.float32) + cb_ref[...]


def _mlp_pool(h, agg, batch2, W1, b1, W2, b2, cls_W, cls_b2):
    return pl.pallas_call(
        _mlp_pool_body,
        grid=(N // BLK,),
        in_specs=[
            pl.BlockSpec((BLK, D), lambda i: (i, 0)),
            pl.BlockSpec((NC, BLK, D), lambda i: (0, i, 0)),
            pl.BlockSpec((BLK, 1), lambda i: (i, 0)),
            pl.BlockSpec((D, H), lambda i: (0, 0)),
            pl.BlockSpec((1, H), lambda i: (0, 0)),
            pl.BlockSpec((H, D), lambda i: (0, 0)),
            pl.BlockSpec((1, D), lambda i: (0, 0)),
            pl.BlockSpec((D, C), lambda i: (0, 0)),
            pl.BlockSpec((1, C), lambda i: (0, 0)),
        ],
        out_specs=pl.BlockSpec((G, C), lambda i: (0, 0)),
        out_shape=jax.ShapeDtypeStruct((G, C), jnp.float32),
        scratch_shapes=[
            pltpu.VMEM((G, D), jnp.float32),
            pltpu.VMEM((G, D), jnp.float32),
        ],
    )(h, agg, batch2, W1, b1, W2, b2, cls_W, cls_b2)


def kernel(x, edge_index, batch,
           l0_W1, l0_b1, l0_bn_g, l0_bn_b, l0_W2, l0_b2, l0_obn_g, l0_obn_b,
           l1_W1, l1_b1, l1_bn_g, l1_bn_b, l1_W2, l1_b2, l1_obn_g, l1_obn_b,
           l2_W1, l2_b1, l2_bn_g, l2_bn_b, l2_W2, l2_b2, l2_obn_g, l2_obn_b,
           cls_W, cls_b):
    bscale = 1.0 / jnp.sqrt(jnp.float32(1.0 + 1e-5))
    layers = [
        (l0_W1, l0_b1, l0_bn_g, l0_bn_b, l0_W2, l0_b2, l0_obn_g, l0_obn_b),
        (l1_W1, l1_b1, l1_bn_g, l1_bn_b, l1_W2, l1_b2, l1_obn_g, l1_obn_b),
        (l2_W1, l2_b1, l2_bn_g, l2_bn_b, l2_W2, l2_b2, l2_obn_g, l2_obn_b),
    ]
    folded = []
    for (W1, b1, bg, bb, W2, b2, og, ob) in layers:
        s1 = bscale * bg
        s2 = bscale * og
        folded.append((W1 * s1[None, :], (b1 * s1 + bb)[None, :],
                       W2 * s2[None, :], (b2 * s2 + ob)[None, :]))

    padw = EPW - E // NW
    pad_src = jnp.broadcast_to(jnp.arange(padw, dtype=jnp.int32), (NW, padw))
    pad_dst = jnp.broadcast_to(N + jnp.arange(padw, dtype=jnp.int32),
                               (NW, padw))
    srcp = jnp.concatenate([edge_index[0].reshape(NW, E // NW), pad_src], 1)
    dstp = jnp.concatenate([edge_index[1].reshape(NW, E // NW), pad_dst], 1)
    packed = ((srcp << PSHIFT) | dstp).reshape(NW, NCHUNK, CHUNK)
    zeros = jnp.zeros((NS, RPT, D), jnp.float32)
    batch2 = batch.reshape(N, 1)

    h = x
    for l in range(2):
        W1f, b1f, W2f, b2f = folded[l]
        agg = _segsum(h, packed, zeros)
        h = _mlp(h, agg, W1f, b1f, W2f, b2f)
    W1f, b1f, W2f, b2f = folded[2]
    agg = _segsum(h, packed, zeros)
    return _mlp_pool(h, agg, batch2, W1f, b1f, W2f, b2f,
                     cls_W, cls_b.reshape(1, C))
